# baseline (speedup 1.0000x reference)
import jax
import jax.numpy as jnp
from jax import lax
from jax.experimental import pallas as pl
from jax.experimental.pallas import tpu as pltpu
from jax.experimental.pallas import tpu_sc as plsc

VOCAB = 100000
EMBED = 64
BATCH = 1024
CTX = 20
HALF = VOCAB // 2
PAIR = 2 * EMBED

NC = 2
NS = 16
NW = NC * NS
IDX_PER_W = BATCH * CTX // NW
GCHUNK = 128
N_CHUNKS = IDX_PER_W // GCHUNK

PACK_R = 2000


def _pack_body(a_ref, b_ref, out_ref):
    out_ref[...] = jnp.concatenate([a_ref[...], b_ref[...]], axis=1)


def _pack(emb):
    return pl.pallas_call(
        _pack_body,
        grid=(HALF // PACK_R,),
        in_specs=[
            pl.BlockSpec((PACK_R, EMBED), lambda j: (j, 0)),
            pl.BlockSpec((PACK_R, EMBED), lambda j: (j + HALF // PACK_R, 0)),
        ],
        out_specs=pl.BlockSpec((PACK_R, PAIR), lambda j: (j, 0)),
        out_shape=jax.ShapeDtypeStruct((HALF, PAIR), jnp.float32),
    )(emb, emb)


def _sc_gather_body(tab_hbm, x_hbm, rows_hbm, idx_v, rows_v, sem):
    wid = lax.axis_index("s") * NC + lax.axis_index("c")
    base = wid * IDX_PER_W
    pltpu.sync_copy(x_hbm.at[pl.ds(base, IDX_PER_W)], idx_v)
    copies = []
    for k in range(N_CHUNKS):
        copies.append(pltpu.async_copy(
            tab_hbm.at[idx_v.at[pl.ds(k * GCHUNK, GCHUNK)]],
            rows_v.at[pl.ds(k * GCHUNK, GCHUNK)],
            sem))
    for cp in copies:
        cp.wait()
    pltpu.sync_copy(rows_v, rows_hbm.at[pl.ds(base, IDX_PER_W)])


def _sc_gather(table2, idx_flat):
    mesh = plsc.VectorSubcoreMesh(core_axis_name="c", subcore_axis_name="s")
    fn = pl.kernel(
        _sc_gather_body,
        mesh=mesh,
        out_type=jax.ShapeDtypeStruct((BATCH * CTX, PAIR), jnp.float32),
        scratch_types=[
            pltpu.VMEM((IDX_PER_W,), jnp.int32),
            pltpu.VMEM((IDX_PER_W, PAIR), jnp.float32),
            pltpu.SemaphoreType.DMA,
        ],
    )
    return fn(table2, idx_flat)


TV = 1024


def _tc_body(rows_ref, x_ref, w_ref, b_ref, out_ref, h_ref):
    @pl.when(pl.program_id(0) == 0)
    def _():
        rows2 = rows_ref[...]
        hi_sel = (x_ref[...] >= HALF).astype(jnp.float32)
        lo = rows2[:, :, :EMBED]
        hi = rows2[:, :, EMBED:]
        rows = lo + hi_sel[:, :, None] * (hi - lo)
        ssq = jnp.sum(rows * rows, axis=-1, keepdims=True)
        norms = jnp.sqrt(ssq)
        scale = jnp.minimum(1.0, 1.0 / jnp.maximum(norms, 1e-7))
        scaled = rows * scale
        acc = scaled[:, 0, :]
        for t in range(1, CTX):
            acc = acc + scaled[:, t, :]
        h_ref[...] = (acc * jnp.float32(1.0 / CTX)).astype(jnp.bfloat16)

    out_ref[...] = lax.dot_general(
        h_ref[...], w_ref[...].astype(jnp.bfloat16),
        dimension_numbers=(((1,), (1,)), ((), ())),
        preferred_element_type=jnp.float32) + b_ref[...]


def _project(rows, x, W, b2d):
    return pl.pallas_call(
        _tc_body,
        grid=(pl.cdiv(VOCAB, TV),),
        in_specs=[
            pl.BlockSpec((BATCH, CTX, PAIR), lambda j: (0, 0, 0)),
            pl.BlockSpec((BATCH, CTX), lambda j: (0, 0)),
            pl.BlockSpec((TV, EMBED), lambda j: (j, 0)),
            pl.BlockSpec((1, TV), lambda j: (0, j)),
        ],
        out_specs=pl.BlockSpec((BATCH, TV), lambda j: (0, j)),
        out_shape=jax.ShapeDtypeStruct((BATCH, VOCAB), jnp.float32),
        scratch_shapes=[pltpu.VMEM((BATCH, EMBED), jnp.bfloat16)],
    )(rows, x, W, b2d)


def kernel(x, emb, W, b):
    table2 = _pack(emb)
    idx = jnp.where(x < HALF, x, x - HALF).reshape(-1)
    rows = _sc_gather(table2, idx)
    return _project(rows.reshape(BATCH, CTX, PAIR), x, W, b.reshape(1, VOCAB))


# speedup vs baseline: 1.0613x; 1.0613x over previous
import jax
import jax.numpy as jnp
from jax import lax
from jax.experimental import pallas as pl
from jax.experimental.pallas import tpu as pltpu
from jax.experimental.pallas import tpu_sc as plsc

VOCAB = 100000
EMBED = 64
BATCH = 1024
CTX = 20
CTXP = 24
HALF = VOCAB // 2
PAIR = 2 * EMBED

NC = 2
NS = 16
NW = NC * NS
B_PER_W = BATCH // NW
ROWS_PER_W = B_PER_W * CTXP


def _sc_gather_body(tab_hbm, idx_hbm, rows_hbm, idx_v, rows_v, sem):
    wid = lax.axis_index("s") * NC + lax.axis_index("c")
    bbase = wid * B_PER_W
    pltpu.sync_copy(idx_hbm.at[pl.ds(bbase, B_PER_W)], idx_v)

    zero = jnp.zeros((16,), jnp.float32)
    def zero_body(i, carry):
        base = i * CTXP + CTX
        for r in range(CTXP - CTX):
            for c in range(PAIR // 16):
                rows_v[base + r, pl.ds(c * 16, 16)] = zero
        return carry
    lax.fori_loop(0, B_PER_W, zero_body, 0)

    copies = []
    for i in range(B_PER_W):
        copies.append(pltpu.async_copy(
            tab_hbm.at[idx_v.at[i]],
            rows_v.at[pl.ds(i * CTXP, CTX)],
            sem))
    for cp in copies:
        cp.wait()
    pltpu.sync_copy(rows_v, rows_hbm.at[pl.ds(wid * ROWS_PER_W, ROWS_PER_W)])


def _sc_gather(table2, idx2d):
    mesh = plsc.VectorSubcoreMesh(core_axis_name="c", subcore_axis_name="s")
    fn = pl.kernel(
        _sc_gather_body,
        mesh=mesh,
        out_type=jax.ShapeDtypeStruct((BATCH * CTXP, PAIR), jnp.float32),
        scratch_types=[
            pltpu.VMEM((B_PER_W, CTX), jnp.int32),
            pltpu.VMEM((ROWS_PER_W, PAIR), jnp.float32),
            pltpu.SemaphoreType.DMA,
        ],
    )
    return fn(table2, idx2d)


TV = 2048


def _tc_body(rows_ref, x_ref, w_ref, b_ref, out_ref, h_ref):
    @pl.when(pl.program_id(0) == 0)
    def _():
        rows2 = rows_ref[...]
        par = (x_ref[...] & 1).astype(jnp.float32)
        par = jnp.concatenate(
            [par, jnp.zeros((BATCH, CTXP - CTX), jnp.float32)], axis=1)
        lo = rows2[:, :, :EMBED]
        hi = rows2[:, :, EMBED:]
        rows = lo + par[:, :, None] * (hi - lo)
        ssq = jnp.sum(rows * rows, axis=-1, keepdims=True)
        norms = jnp.sqrt(ssq)
        scale = jnp.minimum(1.0, 1.0 / jnp.maximum(norms, 1e-7))
        scaled = rows * scale
        acc = scaled[:, 0, :]
        for t in range(1, CTXP):
            acc = acc + scaled[:, t, :]
        h_ref[...] = (acc * jnp.float32(1.0 / CTX)).astype(jnp.bfloat16)

    out_ref[...] = lax.dot_general(
        h_ref[...], w_ref[...].astype(jnp.bfloat16),
        dimension_numbers=(((1,), (1,)), ((), ())),
        preferred_element_type=jnp.float32) + b_ref[...][None, :]


def _project(rows, x, W, b):
    return pl.pallas_call(
        _tc_body,
        grid=(pl.cdiv(VOCAB, TV),),
        in_specs=[
            pl.BlockSpec((BATCH, CTXP, PAIR), lambda j: (0, 0, 0)),
            pl.BlockSpec((BATCH, CTX), lambda j: (0, 0)),
            pl.BlockSpec((TV, EMBED), lambda j: (j, 0)),
            pl.BlockSpec((TV,), lambda j: (j,)),
        ],
        out_specs=pl.BlockSpec((BATCH, TV), lambda j: (0, j)),
        out_shape=jax.ShapeDtypeStruct((BATCH, VOCAB), jnp.float32),
        scratch_shapes=[pltpu.VMEM((BATCH, EMBED), jnp.bfloat16)],
    )(rows, x, W, b)


def kernel(x, emb, W, b):
    table2 = emb.reshape(HALF, PAIR)
    idx2d = x >> 1
    rows = _sc_gather(table2, idx2d)
    return _project(rows.reshape(BATCH, CTXP, PAIR), x, W, b)


# speedup vs baseline: 2.8385x; 2.6747x over previous
import jax
import jax.numpy as jnp
from jax import lax
from jax.experimental import pallas as pl
from jax.experimental.pallas import tpu as pltpu
from jax.experimental.pallas import tpu_sc as plsc

VOCAB = 100000
EMBED = 64
BATCH = 1024
CTX = 20
HALF = VOCAB // 2
PAIR = 2 * EMBED

NC = 2
NS = 16
NW = NC * NS
B_PER_W = BATCH // NW


def _sc_gather_body(tab_hbm, idx_hbm, rows_hbm, idx_v, rows_v, sem):
    wid = lax.axis_index("s") * NC + lax.axis_index("c")
    bbase = wid * B_PER_W
    lane0 = (wid // 4) * 128
    sub = (wid % 4) * B_PER_W
    pltpu.sync_copy(idx_hbm.at[pl.ds(0, CTX), pl.ds(lane0, 128)], idx_v)
    copies = []
    for t in range(CTX):
        copies.append(pltpu.async_copy(
            tab_hbm.at[idx_v.at[t, pl.ds(sub, B_PER_W)]],
            rows_v.at[t],
            sem))
    for cp in copies:
        cp.wait()
    pltpu.sync_copy(rows_v,
                    rows_hbm.at[pl.ds(0, CTX), pl.ds(bbase, B_PER_W)])


def _sc_gather(table2, idxT):
    mesh = plsc.VectorSubcoreMesh(core_axis_name="c", subcore_axis_name="s")
    fn = pl.kernel(
        _sc_gather_body,
        mesh=mesh,
        out_type=jax.ShapeDtypeStruct((CTX, BATCH, PAIR), jnp.float32),
        scratch_types=[
            pltpu.VMEM((CTX, 128), jnp.int32),
            pltpu.VMEM((CTX, B_PER_W, PAIR), jnp.float32),
            pltpu.SemaphoreType.DMA,
        ],
    )
    return fn(table2, idxT)


TV = 2048


def _tc_body(rows_ref, x_ref, w_ref, b_ref, out_ref, h_ref):
    @pl.when(pl.program_id(0) == 0)
    def _():
        rows2 = rows_ref[...]
        par = (x_ref[...] & 1).astype(jnp.float32)
        lo = rows2[:, :, :EMBED]
        hi = rows2[:, :, EMBED:]
        rows = lo + par[:, :, None] * (hi - lo)
        ssq = jnp.sum(rows * rows, axis=-1, keepdims=True)
        norms = jnp.sqrt(ssq)
        scale = jnp.minimum(1.0, 1.0 / jnp.maximum(norms, 1e-7))
        scaled = rows * scale
        acc = scaled[0]
        for t in range(1, CTX):
            acc = acc + scaled[t]
        h = acc * jnp.float32(1.0 / CTX)
        h_ref[...] = h.T.astype(jnp.bfloat16)

    outT = lax.dot_general(
        w_ref[...].astype(jnp.bfloat16), h_ref[...],
        dimension_numbers=(((0,), (0,)), ((), ())),
        preferred_element_type=jnp.float32)
    out_ref[...] = outT + b_ref[...][:, None]


def _project(rowsT, xT, WT, b):
    return pl.pallas_call(
        _tc_body,
        grid=(pl.cdiv(VOCAB, TV),),
        in_specs=[
            pl.BlockSpec((CTX, BATCH, PAIR), lambda j: (0, 0, 0)),
            pl.BlockSpec((CTX, BATCH), lambda j: (0, 0)),
            pl.BlockSpec((EMBED, TV), lambda j: (0, j)),
            pl.BlockSpec((TV,), lambda j: (j,)),
        ],
        out_specs=pl.BlockSpec((TV, BATCH), lambda j: (j, 0)),
        out_shape=jax.ShapeDtypeStruct((VOCAB, BATCH), jnp.float32),
        scratch_shapes=[pltpu.VMEM((EMBED, BATCH), jnp.bfloat16)],
    )(rowsT, xT, WT, b)


def kernel(x, emb, W, b):
    xT = x.T
    idxT = xT >> 1
    table2 = emb.reshape(HALF, PAIR)
    rowsT = _sc_gather(table2, idxT)
    outT = _project(rowsT, xT, W.T, b)
    return outT.T


# speedup vs baseline: 3.1639x; 1.1146x over previous
import jax
import jax.numpy as jnp
from jax import lax
from jax.experimental import pallas as pl
from jax.experimental.pallas import tpu as pltpu
from jax.experimental.pallas import tpu_sc as plsc

VOCAB = 100000
EMBED = 64
BATCH = 1024
CTX = 20
PAIR = 2 * EMBED

NC = 2
NS = 16
NW = NC * NS
B_PER_W = BATCH // NW

PACK_TV = 2048


def _pack_body(embT_ref, out_ref):
    blk = embT_ref[...]
    rows = blk.T
    out_ref[...] = jnp.concatenate(
        [rows, jnp.zeros((PACK_TV, PAIR - EMBED), jnp.float32)], axis=1)


def _pack(embT):
    return pl.pallas_call(
        _pack_body,
        grid=(pl.cdiv(VOCAB, PACK_TV),),
        in_specs=[pl.BlockSpec((EMBED, PACK_TV), lambda j: (0, j))],
        out_specs=pl.BlockSpec((PACK_TV, PAIR), lambda j: (j, 0)),
        out_shape=jax.ShapeDtypeStruct((VOCAB, PAIR), jnp.float32),
    )(embT)


def _sc_gather_body(tab_hbm, idx_hbm, rows_hbm, idx_v, rows_v, sem):
    wid = lax.axis_index("s") * NC + lax.axis_index("c")
    bbase = wid * B_PER_W
    lane0 = (wid // 4) * 128
    sub = (wid % 4) * B_PER_W
    pltpu.sync_copy(idx_hbm.at[pl.ds(0, CTX), pl.ds(lane0, 128)], idx_v)
    copies = []
    for t in range(CTX):
        copies.append(pltpu.async_copy(
            tab_hbm.at[idx_v.at[t, pl.ds(sub, B_PER_W)]],
            rows_v.at[t],
            sem))
    for cp in copies:
        cp.wait()
    pltpu.sync_copy(rows_v,
                    rows_hbm.at[pl.ds(0, CTX), pl.ds(bbase, B_PER_W)])


def _sc_gather(table, idxT):
    mesh = plsc.VectorSubcoreMesh(core_axis_name="c", subcore_axis_name="s")
    fn = pl.kernel(
        _sc_gather_body,
        mesh=mesh,
        out_type=jax.ShapeDtypeStruct((CTX, BATCH, PAIR), jnp.float32),
        scratch_types=[
            pltpu.VMEM((CTX, 128), jnp.int32),
            pltpu.VMEM((CTX, B_PER_W, PAIR), jnp.float32),
            pltpu.SemaphoreType.DMA,
        ],
    )
    return fn(table, idxT)


TV = 2048


def _tc_body(rows_ref, w_ref, b_ref, out_ref, h_ref):
    @pl.when(pl.program_id(0) == 0)
    def _():
        rows = rows_ref[...][:, :, :EMBED]
        ssq = jnp.sum(rows * rows, axis=-1, keepdims=True)
        norms = jnp.sqrt(ssq)
        scale = jnp.minimum(1.0, 1.0 / jnp.maximum(norms, 1e-7))
        scaled = rows * scale
        acc = scaled[0]
        for t in range(1, CTX):
            acc = acc + scaled[t]
        h = acc * jnp.float32(1.0 / CTX)
        h_ref[...] = h.T.astype(jnp.bfloat16)

    outT = lax.dot_general(
        w_ref[...].astype(jnp.bfloat16), h_ref[...],
        dimension_numbers=(((0,), (0,)), ((), ())),
        preferred_element_type=jnp.float32)
    out_ref[...] = outT + b_ref[...][:, None]


def _project(rowsT, WT, b):
    return pl.pallas_call(
        _tc_body,
        grid=(pl.cdiv(VOCAB, TV),),
        in_specs=[
            pl.BlockSpec((CTX, BATCH, PAIR), lambda j: (0, 0, 0)),
            pl.BlockSpec((EMBED, TV), lambda j: (0, j)),
            pl.BlockSpec((TV,), lambda j: (j,)),
        ],
        out_specs=pl.BlockSpec((TV, BATCH), lambda j: (j, 0)),
        out_shape=jax.ShapeDtypeStruct((VOCAB, BATCH), jnp.float32),
        scratch_shapes=[pltpu.VMEM((EMBED, BATCH), jnp.bfloat16)],
    )(rowsT, WT, b)


def kernel(x, emb, W, b):
    table = _pack(emb.T)
    rowsT = _sc_gather(table, x.T)
    outT = _project(rowsT, W.T, b)
    return outT.T


# speedup vs baseline: 3.5095x; 1.1092x over previous
import jax
import jax.numpy as jnp
from jax import lax
from jax.experimental import pallas as pl
from jax.experimental.pallas import tpu as pltpu
from jax.experimental.pallas import tpu_sc as plsc

VOCAB = 100000
EMBED = 64
BATCH = 1024
CTX = 20
PAIR = 2 * EMBED

NC = 2
NS = 16
NW = NC * NS
B_PER_W = BATCH // NW

PACK_TV = 8192


def _pack_body(embT_ref, out_ref):
    blk = embT_ref[...]
    rows = blk.T
    out_ref[...] = jnp.concatenate(
        [rows, jnp.zeros((PACK_TV, PAIR - EMBED), jnp.float32)], axis=1)


def _pack(embT):
    return pl.pallas_call(
        _pack_body,
        grid=(pl.cdiv(VOCAB, PACK_TV),),
        in_specs=[pl.BlockSpec((EMBED, PACK_TV), lambda j: (0, j))],
        out_specs=pl.BlockSpec((PACK_TV, PAIR), lambda j: (j, 0)),
        out_shape=jax.ShapeDtypeStruct((VOCAB, PAIR), jnp.float32),
    )(embT)


def _sc_gather_body(tab_hbm, idx_hbm, rows_hbm, idx_v, rows_v, sem):
    wid = lax.axis_index("s") * NC + lax.axis_index("c")
    bbase = wid * B_PER_W
    lane0 = (wid // 4) * 128
    sub = (wid % 4) * B_PER_W
    pltpu.sync_copy(idx_hbm.at[pl.ds(0, CTX), pl.ds(lane0, 128)], idx_v)
    copies = []
    for t in range(CTX):
        copies.append(pltpu.async_copy(
            tab_hbm.at[idx_v.at[t, pl.ds(sub, B_PER_W)]],
            rows_v.at[t],
            sem))
    for cp in copies:
        cp.wait()
    pltpu.sync_copy(rows_v,
                    rows_hbm.at[pl.ds(0, CTX), pl.ds(bbase, B_PER_W)])


def _sc_gather(table, idxT):
    mesh = plsc.VectorSubcoreMesh(core_axis_name="c", subcore_axis_name="s")
    fn = pl.kernel(
        _sc_gather_body,
        mesh=mesh,
        out_type=jax.ShapeDtypeStruct((CTX, BATCH, PAIR), jnp.float32),
        scratch_types=[
            pltpu.VMEM((CTX, 128), jnp.int32),
            pltpu.VMEM((CTX, B_PER_W, PAIR), jnp.float32),
            pltpu.SemaphoreType.DMA,
        ],
    )
    return fn(table, idxT)


TV = 4096


def _tc_body(rows_ref, w_ref, b_ref, out_ref, h_ref):
    @pl.when(pl.program_id(0) == 0)
    def _():
        rows = rows_ref[...][:, :, :EMBED]
        ssq = jnp.sum(rows * rows, axis=-1, keepdims=True)
        norms = jnp.sqrt(ssq)
        scale = jnp.minimum(1.0, 1.0 / jnp.maximum(norms, 1e-7))
        scaled = rows * scale
        acc = scaled[0]
        for t in range(1, CTX):
            acc = acc + scaled[t]
        h = acc * jnp.float32(1.0 / CTX)
        h_ref[...] = h.T.astype(jnp.bfloat16)

    outT = lax.dot_general(
        w_ref[...].astype(jnp.bfloat16), h_ref[...],
        dimension_numbers=(((0,), (0,)), ((), ())),
        preferred_element_type=jnp.float32)
    out_ref[...] = outT + b_ref[...][:, None]


def _project(rowsT, WT, b):
    return pl.pallas_call(
        _tc_body,
        grid=(pl.cdiv(VOCAB, TV),),
        in_specs=[
            pl.BlockSpec((CTX, BATCH, PAIR), lambda j: (0, 0, 0)),
            pl.BlockSpec((EMBED, TV), lambda j: (0, j)),
            pl.BlockSpec((TV,), lambda j: (j,)),
        ],
        out_specs=pl.BlockSpec((TV, BATCH), lambda j: (j, 0)),
        out_shape=jax.ShapeDtypeStruct((VOCAB, BATCH), jnp.float32),
        scratch_shapes=[pltpu.VMEM((EMBED, BATCH), jnp.bfloat16)],
    )(rowsT, WT, b)


def kernel(x, emb, W, b):
    table = _pack(emb.T)
    rowsT = _sc_gather(table, x.T)
    outT = _project(rowsT, W.T, b)
    return outT.T
